# Initial kernel scaffold; baseline (speedup 1.0000x reference)
import jax
import jax.numpy as jnp
from jax.experimental import pallas as pl


def kernel(token_ids, weight):
    raise NotImplementedError("write your pallas kernel here")


# speedup vs baseline: 1.7934x; 1.7934x over previous
import functools

import jax
import jax.numpy as jnp
from jax import lax
from jax.experimental import pallas as pl
from jax.experimental.pallas import tpu as pltpu
from jax.experimental.pallas import tpu_sc as plsc

D = 64
B = 16384 * 50
NC, NS = 2, 16
NW = NC * NS
B_PER_W = B // NW
CHUNK = 128
N_CHUNKS = B_PER_W // CHUNK

_mesh = plsc.VectorSubcoreMesh(core_axis_name="c", subcore_axis_name="s")


@functools.partial(
    pl.kernel,
    mesh=_mesh,
    compiler_params=pltpu.CompilerParams(use_tc_tiling_on_sc=False),
    out_type=jax.ShapeDtypeStruct((B, D), jnp.float32),
    scratch_types=[
        pltpu.VMEM((N_CHUNKS, CHUNK), jnp.int32),
        pltpu.VMEM((CHUNK, D), jnp.float32),
        pltpu.VMEM((CHUNK, D), jnp.float32),
        pltpu.SemaphoreType.DMA,
        pltpu.SemaphoreType.DMA,
    ],
)
def _emb_lookup(idx_hbm, tab_hbm, out_hbm, idx_v, rows0, rows1, sem0, sem1):
    wid = lax.axis_index("s") * NC + lax.axis_index("c")
    base = wid * B_PER_W
    pltpu.sync_copy(idx_hbm.at[wid], idx_v)

    def body(g, carry):
        j0 = g * 2
        j1 = j0 + 1
        c0 = pltpu.async_copy(tab_hbm.at[idx_v.at[j0]], rows0, sem0)
        c1 = pltpu.async_copy(tab_hbm.at[idx_v.at[j1]], rows1, sem1)
        c0.wait()
        pltpu.sync_copy(rows0, out_hbm.at[pl.ds(base + j0 * CHUNK, CHUNK)])
        c1.wait()
        pltpu.sync_copy(rows1, out_hbm.at[pl.ds(base + j1 * CHUNK, CHUNK)])
        return carry

    lax.fori_loop(0, N_CHUNKS // 2, body, 0)


def kernel(token_ids, weight):
    idx = token_ids.reshape(NW, N_CHUNKS, CHUNK).astype(jnp.int32)
    out = _emb_lookup(idx, weight)
    return out.reshape(token_ids.shape + (D,))


# speedup vs baseline: 1.8740x; 1.0450x over previous
import functools

import jax
import jax.numpy as jnp
from jax import lax
from jax.experimental import pallas as pl
from jax.experimental.pallas import tpu as pltpu
from jax.experimental.pallas import tpu_sc as plsc

D = 64
B = 16384 * 50
NC, NS = 2, 16
NW = NC * NS
B_PER_W = B // NW
CHUNK = 128
WCHUNK = 256
SPB = WCHUNK // CHUNK
N_SLOTS = B_PER_W // WCHUNK
NBUF = 5
LAG = 2

_mesh = plsc.VectorSubcoreMesh(core_axis_name="c", subcore_axis_name="s")


@functools.partial(
    pl.kernel,
    mesh=_mesh,
    compiler_params=pltpu.CompilerParams(use_tc_tiling_on_sc=False),
    out_type=jax.ShapeDtypeStruct((B, D), jnp.float32),
    scratch_types=(
        [pltpu.VMEM((B_PER_W // CHUNK, CHUNK), jnp.int32)]
        + [pltpu.VMEM((WCHUNK, D), jnp.float32) for _ in range(NBUF)]
        + [pltpu.SemaphoreType.DMA for _ in range(2 * NBUF)]
    ),
)
def _emb_lookup(idx_hbm, tab_hbm, out_hbm, idx_v, *scratch):
    rows = scratch[:NBUF]
    gsem = scratch[NBUF:2 * NBUF]
    wsem = scratch[2 * NBUF:]
    wid = lax.axis_index("s") * NC + lax.axis_index("c")
    base = wid * B_PER_W
    pltpu.sync_copy(idx_hbm.at[wid], idx_v)

    def g_issue(j, b):
        for h in range(SPB):
            pltpu.async_copy(
                tab_hbm.at[idx_v.at[j * SPB + h]],
                rows[b].at[pl.ds(h * CHUNK, CHUNK)],
                gsem[b],
            )

    def g_wait(b):
        pltpu.make_async_copy(out_hbm.at[pl.ds(0, WCHUNK)], rows[b], gsem[b]).wait()

    def w_issue(j, b):
        pltpu.async_copy(rows[b], out_hbm.at[pl.ds(base + j * WCHUNK, WCHUNK)], wsem[b])

    def w_wait(b):
        pltpu.make_async_copy(rows[b], out_hbm.at[pl.ds(0, WCHUNK)], wsem[b]).wait()

    def group(gi, carry):
        for b in range(NBUF):
            j = gi * NBUF + b

            @pl.when(j < N_SLOTS)
            def _():
                @pl.when(j >= NBUF)
                def _():
                    w_wait(b)
                g_issue(j, b)

            jw = j - LAG
            bw = (b - LAG) % NBUF

            @pl.when((jw >= 0) & (jw < N_SLOTS))
            def _():
                g_wait(bw)
                w_issue(jw, bw)

        return carry

    n_groups = (N_SLOTS + LAG + NBUF - 1) // NBUF
    lax.fori_loop(0, n_groups, group, 0)
    for b in range(NBUF):
        w_wait(b)


def kernel(token_ids, weight):
    idx = token_ids.reshape(NW, B_PER_W // CHUNK, CHUNK).astype(jnp.int32)
    out = _emb_lookup(idx, weight)
    return out.reshape(token_ids.shape + (D,))
